# baseline (speedup 1.0000x reference)
import functools

import jax
import jax.numpy as jnp
from jax.experimental import pallas as pl
from jax.experimental.pallas import tpu as pltpu

NUM_TYPES = 10
N = 100000
R = 8


def _kernel(act_ref, cum_ref, x_ref, lp_ref, sel_ref, ent_ref):
    i = pl.program_id(0)
    x = x_ref[...]
    m = jnp.max(x, axis=1, keepdims=True)
    s = x - m
    e = jnp.exp(s)
    ones = jnp.ones((N, 1), dtype=jnp.float32)
    z = jax.lax.dot_general(
        e, ones, (((1,), (0,)), ((), ())),
        precision=jax.lax.Precision.DEFAULT,
        preferred_element_type=jnp.float32,
    )
    es = jnp.sum(e * s, axis=1, keepdims=True)
    lz = jnp.log(z)
    lp_ref[...] = s - lz
    ent_ref[...] = lz - es / z

    for r in range(R):
        row = i * R + r
        at = act_ref[row, 0]
        ap = act_ref[row, 1]
        idx = cum_ref[at] + at + ap
        base = pl.multiple_of((idx // 128) * 128, 128)
        lane = idx - base
        chunk = x_ref[r : r + 1, pl.ds(base, 128)]
        lix = jax.lax.broadcasted_iota(jnp.int32, (1, 128), 1)
        v = jnp.sum(jnp.where(lix == lane, chunk, 0.0), axis=1, keepdims=True)
        sel_ref[r : r + 1, :] = v - m[r : r + 1, :] - lz[r : r + 1, :]


@jax.jit
def kernel(logits, action, cum_action_max_params):
    b = logits.shape[0]
    grid = (b // R,)
    lp, sel, ent = pl.pallas_call(
        _kernel,
        grid=grid,
        in_specs=[
            pl.BlockSpec(memory_space=pltpu.SMEM),
            pl.BlockSpec(memory_space=pltpu.SMEM),
            pl.BlockSpec((R, N), lambda i: (i, 0)),
        ],
        out_specs=[
            pl.BlockSpec((R, N), lambda i: (i, 0)),
            pl.BlockSpec((R, 1), lambda i: (i, 0)),
            pl.BlockSpec((R, 1), lambda i: (i, 0)),
        ],
        out_shape=[
            jax.ShapeDtypeStruct((b, N), jnp.float32),
            jax.ShapeDtypeStruct((b, 1), jnp.float32),
            jax.ShapeDtypeStruct((b, 1), jnp.float32),
        ],
        compiler_params=pltpu.CompilerParams(
            dimension_semantics=("arbitrary",),
        ),
    )(action.astype(jnp.int32), cum_action_max_params, logits)
    return sel[:, 0], ent[:, 0], lp


# speedup vs baseline: 1.1789x; 1.1789x over previous
import functools

import jax
import jax.numpy as jnp
from jax.experimental import pallas as pl
from jax.experimental.pallas import tpu as pltpu

NUM_TYPES = 10
N = 100000
R = 8
CW = 1024
NCH = N // CW
TAIL = N - NCH * CW
TBASE = NCH * CW


def _kernel(act_ref, cum_ref, x_ref, lp_ref, sel_ref, ent_ref):
    i = pl.program_id(0)
    m = jnp.max(x_ref[...], axis=1, keepdims=True)

    def body(c, carry):
        z_acc, es_acc = carry
        xv = x_ref[:, pl.ds(c * CW, CW)]
        s = xv - m
        e = jnp.exp(s)
        return z_acc + e, es_acc + e * s

    z_acc, es_acc = jax.lax.fori_loop(
        0,
        NCH,
        body,
        (jnp.zeros((R, CW), jnp.float32), jnp.zeros((R, CW), jnp.float32)),
    )
    st = x_ref[:, pl.ds(TBASE, TAIL)] - m
    et = jnp.exp(st)
    z = jnp.sum(z_acc, axis=1, keepdims=True) + jnp.sum(et, axis=1, keepdims=True)
    es = jnp.sum(es_acc, axis=1, keepdims=True) + jnp.sum(
        et * st, axis=1, keepdims=True
    )
    lz = jnp.log(z)
    mlz = m + lz
    lp_ref[...] = x_ref[...] - mlz
    ent_ref[...] = lz - es / z

    for r in range(R):
        row = i * R + r
        at = act_ref[row, 0]
        ap = act_ref[row, 1]
        idx = cum_ref[at] + at + ap
        base = pl.multiple_of((idx // 128) * 128, 128)
        lane = idx - base
        chunk = x_ref[r : r + 1, pl.ds(base, 128)]
        lix = jax.lax.broadcasted_iota(jnp.int32, (1, 128), 1)
        v = jnp.sum(jnp.where(lix == lane, chunk, 0.0), axis=1, keepdims=True)
        sel_ref[r : r + 1, :] = v - mlz[r : r + 1, :]


@jax.jit
def kernel(logits, action, cum_action_max_params):
    b = logits.shape[0]
    grid = (b // R,)
    lp, sel, ent = pl.pallas_call(
        _kernel,
        grid=grid,
        in_specs=[
            pl.BlockSpec(memory_space=pltpu.SMEM),
            pl.BlockSpec(memory_space=pltpu.SMEM),
            pl.BlockSpec((R, N), lambda i: (i, 0)),
        ],
        out_specs=[
            pl.BlockSpec((R, N), lambda i: (i, 0)),
            pl.BlockSpec((R, 1), lambda i: (i, 0)),
            pl.BlockSpec((R, 1), lambda i: (i, 0)),
        ],
        out_shape=[
            jax.ShapeDtypeStruct((b, N), jnp.float32),
            jax.ShapeDtypeStruct((b, 1), jnp.float32),
            jax.ShapeDtypeStruct((b, 1), jnp.float32),
        ],
        compiler_params=pltpu.CompilerParams(
            dimension_semantics=("arbitrary",),
        ),
    )(action.astype(jnp.int32), cum_action_max_params, logits)
    return sel[:, 0], ent[:, 0], lp


# speedup vs baseline: 1.2990x; 1.1018x over previous
import functools

import jax
import jax.numpy as jnp
from jax.experimental import pallas as pl
from jax.experimental.pallas import tpu as pltpu

NUM_TYPES = 10
N = 100000
R = 16
CW = 1024
NCH = N // CW
TAIL = N - NCH * CW
TBASE = NCH * CW


def _kernel(act_ref, cum_ref, x_ref, lp_ref, sel_ref, ent_ref):
    i = pl.program_id(0)
    m = jnp.max(x_ref[...], axis=1, keepdims=True)

    def body(c, carry):
        z_acc, es_acc = carry
        xv = x_ref[:, pl.ds(c * CW, CW)]
        s = xv - m
        e = jnp.exp(s)
        return z_acc + e, es_acc + e * s

    z_acc, es_acc = jax.lax.fori_loop(
        0,
        NCH,
        body,
        (jnp.zeros((R, CW), jnp.float32), jnp.zeros((R, CW), jnp.float32)),
    )
    st = x_ref[:, pl.ds(TBASE, TAIL)] - m
    et = jnp.exp(st)
    z = jnp.sum(z_acc, axis=1, keepdims=True) + jnp.sum(et, axis=1, keepdims=True)
    es = jnp.sum(es_acc, axis=1, keepdims=True) + jnp.sum(
        et * st, axis=1, keepdims=True
    )
    lz = jnp.log(z)
    mlz = m + lz
    lp_ref[...] = x_ref[...] - mlz
    ent_ref[...] = lz - es / z

    for r in range(R):
        row = i * R + r
        at = act_ref[row, 0]
        ap = act_ref[row, 1]
        idx = cum_ref[at] + at + ap
        base = pl.multiple_of((idx // 128) * 128, 128)
        lane = idx - base
        chunk = x_ref[r : r + 1, pl.ds(base, 128)]
        lix = jax.lax.broadcasted_iota(jnp.int32, (1, 128), 1)
        v = jnp.sum(jnp.where(lix == lane, chunk, 0.0), axis=1, keepdims=True)
        sel_ref[r : r + 1, :] = v - mlz[r : r + 1, :]


@jax.jit
def kernel(logits, action, cum_action_max_params):
    b = logits.shape[0]
    grid = (b // R,)
    lp, sel, ent = pl.pallas_call(
        _kernel,
        grid=grid,
        in_specs=[
            pl.BlockSpec(memory_space=pltpu.SMEM),
            pl.BlockSpec(memory_space=pltpu.SMEM),
            pl.BlockSpec((R, N), lambda i: (i, 0)),
        ],
        out_specs=[
            pl.BlockSpec((R, N), lambda i: (i, 0)),
            pl.BlockSpec((R, 1), lambda i: (i, 0)),
            pl.BlockSpec((R, 1), lambda i: (i, 0)),
        ],
        out_shape=[
            jax.ShapeDtypeStruct((b, N), jnp.float32),
            jax.ShapeDtypeStruct((b, 1), jnp.float32),
            jax.ShapeDtypeStruct((b, 1), jnp.float32),
        ],
        compiler_params=pltpu.CompilerParams(
            dimension_semantics=("arbitrary",),
        ),
    )(action.astype(jnp.int32), cum_action_max_params, logits)
    return sel[:, 0], ent[:, 0], lp


# speedup vs baseline: 1.2996x; 1.0005x over previous
import functools

import jax
import jax.numpy as jnp
from jax.experimental import pallas as pl
from jax.experimental.pallas import tpu as pltpu

NUM_TYPES = 10
N = 100000
R = 16
CW = 1024
NCH = N // CW
TAIL = N - NCH * CW
TBASE = NCH * CW


def _kernel(act_ref, cum_ref, x_ref, lp_ref, sel_ref, ent_ref):
    i = pl.program_id(0)
    m = jnp.max(x_ref[...], axis=1, keepdims=True)

    def body(c, carry):
        z_acc, es_acc = carry
        xv = x_ref[:, pl.ds(c * CW, CW)]
        s = xv - m
        e = jnp.exp(s)
        return z_acc + e, es_acc + e * s

    z_acc, es_acc = jax.lax.fori_loop(
        0,
        NCH,
        body,
        (jnp.zeros((R, CW), jnp.float32), jnp.zeros((R, CW), jnp.float32)),
    )
    st = x_ref[:, pl.ds(TBASE, TAIL)] - m
    et = jnp.exp(st)
    z = jnp.sum(z_acc, axis=1, keepdims=True) + jnp.sum(et, axis=1, keepdims=True)
    es = jnp.sum(es_acc, axis=1, keepdims=True) + jnp.sum(
        et * st, axis=1, keepdims=True
    )
    lz = jnp.log(z)
    mlz = m + lz
    lp_ref[...] = x_ref[...] - mlz
    ent_ref[...] = lz - es / z

    for r in range(R):
        row = i * R + r
        at = act_ref[row, 0]
        ap = act_ref[row, 1]
        idx = cum_ref[at] + at + ap
        base = pl.multiple_of((idx // 128) * 128, 128)
        lane = idx - base
        chunk = x_ref[r : r + 1, pl.ds(base, 128)]
        lix = jax.lax.broadcasted_iota(jnp.int32, (1, 128), 1)
        v = jnp.sum(jnp.where(lix == lane, chunk, 0.0), axis=1, keepdims=True)
        sel_ref[r : r + 1, :] = v - mlz[r : r + 1, :]


@jax.jit
def kernel(logits, action, cum_action_max_params):
    b = logits.shape[0]
    grid = (b // R,)
    lp, sel, ent = pl.pallas_call(
        _kernel,
        grid=grid,
        in_specs=[
            pl.BlockSpec(memory_space=pltpu.SMEM),
            pl.BlockSpec(memory_space=pltpu.SMEM),
            pl.BlockSpec((R, N), lambda i: (i, 0)),
        ],
        out_specs=[
            pl.BlockSpec((R, N), lambda i: (i, 0)),
            pl.BlockSpec((R, 1), lambda i: (i, 0)),
            pl.BlockSpec((R, 1), lambda i: (i, 0)),
        ],
        out_shape=[
            jax.ShapeDtypeStruct((b, N), jnp.float32),
            jax.ShapeDtypeStruct((b, 1), jnp.float32),
            jax.ShapeDtypeStruct((b, 1), jnp.float32),
        ],
        compiler_params=pltpu.CompilerParams(
            dimension_semantics=("parallel",),
        ),
    )(action.astype(jnp.int32), cum_action_max_params, logits)
    return sel[:, 0], ent[:, 0], lp
